# Initial kernel scaffold; baseline (speedup 1.0000x reference)
import jax
import jax.numpy as jnp
from jax.experimental import pallas as pl


def kernel(x_nchw, w1, t1, w2, t2, wf1, t3, wf2, b2):
    raise NotImplementedError("write your pallas kernel here")


# speedup vs baseline: 23.1683x; 23.1683x over previous
import jax
import jax.numpy as jnp
from jax.experimental import pallas as pl
from jax.experimental.pallas import tpu as pltpu


def _fused_cnn_kernel(x_ref, T1_ref, t1m_ref, T2_ref, t2m_ref,
                      wf1_ref, t3_ref, wf2_ref, b2_ref, o_ref,
                      y1s_ref, ph_ref, y2s_ref):
    tb = x_ref.shape[0]

    A1 = x_ref[...].reshape(tb * 32, 32)
    B1 = jnp.dot(A1, T1_ref[...], preferred_element_type=jnp.float32)
    B1 = B1.reshape(tb, 32, 768)
    y1 = (B1[:, 0:28, 0:256] + B1[:, 1:29, 256:512] + B1[:, 2:30, 512:768])
    y1s_ref[...] = jnp.maximum(y1 + t1m_ref[...], 0.0)

    ph_ref[:, 0, :] = jnp.zeros((tb, 128), jnp.float32)
    ph_ref[:, 15, :] = jnp.zeros((tb, 128), jnp.float32)
    for q in range(14):
        m = jnp.maximum(y1s_ref[:, 2 * q, :], y1s_ref[:, 2 * q + 1, :])
        ph_ref[:, q + 1, :] = jnp.maximum(m[:, 0:128], m[:, 128:256])

    A2 = ph_ref[...].reshape(tb * 16, 128)
    B2 = jnp.dot(A2, T2_ref[...], preferred_element_type=jnp.float32)
    B2 = B2.reshape(tb, 16, 3072)
    y2 = (B2[:, 0:14, 0:1024] + B2[:, 1:15, 1024:2048]
          + B2[:, 2:16, 2048:3072])
    y2s_ref[...] = jnp.maximum(y2 + t2m_ref[...], 0.0)

    acc = jnp.zeros((tb, 128), jnp.float32)
    for h in range(7):
        m = jnp.maximum(y2s_ref[:, 2 * h, :], y2s_ref[:, 2 * h + 1, :])
        slab = jnp.maximum(m[:, 0:512], m[:, 512:1024])
        acc = acc + jnp.dot(slab, wf1_ref[h],
                            preferred_element_type=jnp.float32)

    h1 = jnp.maximum(acc + t3_ref[...], 0.0)
    out = jnp.dot(h1, wf2_ref[...], preferred_element_type=jnp.float32)
    o_ref[...] = out + b2_ref[...]


def _pick_tile(n, candidates):
    for c in candidates:
        if n % c == 0:
            return c
    return n


def _build_tables(w1, t1, w2, t2):
    f32 = jnp.float32
    kh, kw, c, p, jj = jnp.meshgrid(jnp.arange(3), jnp.arange(3),
                                    jnp.arange(6), jnp.arange(2),
                                    jnp.arange(14), indexing="ij")
    rows = 2 * jj + p + kw
    cols = kh * 256 + p * 128 + c * 16 + (jj + 1)
    vals = w1[kh * 3 + kw, c]
    T1 = jnp.zeros((32, 768), f32).at[rows, cols].set(vals)

    c1, p1, jj1 = jnp.meshgrid(jnp.arange(6), jnp.arange(2),
                               jnp.arange(14), indexing="ij")
    t1m = jnp.zeros((1, 256), f32).at[0, p1 * 128 + c1 * 16 + jj1 + 1].set(
        jnp.broadcast_to(t1[0, c1], c1.shape))

    kh2, kw2, c2, p2, jj2 = jnp.meshgrid(jnp.arange(3), jnp.arange(3),
                                         jnp.arange(6), jnp.arange(2),
                                         jnp.arange(7), indexing="ij")
    rows2 = (c2 * 16 + 2 * jj2 + p2 + kw2)[..., None]
    cols2 = (kh2 * 1024 + p2 * 512 + jj2 * 64)[..., None] + jnp.arange(64)
    vals2 = w2[(kh2 * 3 + kw2) * 6 + c2, :]
    T2 = jnp.zeros((128, 3072), f32).at[
        jnp.broadcast_to(rows2, vals2.shape),
        jnp.broadcast_to(cols2, vals2.shape)].set(vals2)

    t2half = jnp.pad(jnp.tile(t2, (1, 7)), ((0, 0), (0, 64)))
    t2m = jnp.tile(t2half, (1, 2)).reshape(1, 1, 1024)
    return T1, t1m.reshape(1, 1, 256), T2, t2m


def kernel(x_nchw, w1, t1, w2, t2, wf1, t3, wf2, b2):
    n = x_nchw.shape[0]
    x = x_nchw.reshape(n, 28, 28)
    xpad = jnp.pad(x, ((0, 0), (1, 3), (1, 3)))
    T1, t1m, T2, t2m = _build_tables(w1, t1, w2, t2)
    wf1r = jnp.pad(wf1.reshape(7, 448, 128), ((0, 0), (0, 64), (0, 0)))

    tb = _pick_tile(n, (32, 16, 8, 4, 2, 1))
    return pl.pallas_call(
        _fused_cnn_kernel,
        out_shape=jax.ShapeDtypeStruct((n, 10), jnp.float32),
        grid=(n // tb,),
        in_specs=[
            pl.BlockSpec((tb, 32, 32), lambda i: (i, 0, 0)),
            pl.BlockSpec((32, 768), lambda i: (0, 0)),
            pl.BlockSpec((1, 1, 256), lambda i: (0, 0, 0)),
            pl.BlockSpec((128, 3072), lambda i: (0, 0)),
            pl.BlockSpec((1, 1, 1024), lambda i: (0, 0, 0)),
            pl.BlockSpec((7, 512, 128), lambda i: (0, 0, 0)),
            pl.BlockSpec((1, 128), lambda i: (0, 0)),
            pl.BlockSpec((128, 10), lambda i: (0, 0)),
            pl.BlockSpec((1, 10), lambda i: (0, 0)),
        ],
        out_specs=pl.BlockSpec((tb, 10), lambda i: (i, 0)),
        scratch_shapes=[
            pltpu.VMEM((tb, 28, 256), jnp.float32),
            pltpu.VMEM((tb, 16, 128), jnp.float32),
            pltpu.VMEM((tb, 14, 1024), jnp.float32),
        ],
        compiler_params=pltpu.CompilerParams(
            dimension_semantics=("parallel",),
            vmem_limit_bytes=100 * 1024 * 1024,
        ),
    )(xpad, T1, t1m, T2, t2m, wf1r, t3, wf2, b2)
